# baseline (speedup 1.0000x reference)
import functools

import jax
import jax.numpy as jnp
from jax import lax
from jax.experimental import pallas as pl
from jax.experimental.pallas import tpu as pltpu

EPS = 1e-5
NC, NS = 2, 16
NW = NC * NS


def _sc_gather(idx_pad, emb_table, D):
    from jax.experimental.pallas import tpu_sc as plsc

    TPAD = idx_pad.shape[0]
    rows_per_w = TPAD // NW
    mesh = plsc.VectorSubcoreMesh(core_axis_name="c", subcore_axis_name="s")

    @functools.partial(
        pl.kernel,
        mesh=mesh,
        compiler_params=pltpu.CompilerParams(use_tc_tiling_on_sc=False),
        out_type=jax.ShapeDtypeStruct((TPAD, D), jnp.float32),
        scratch_types=[
            pltpu.VMEM((rows_per_w,), jnp.int32),
            pltpu.VMEM((rows_per_w, D), jnp.float32),
            pltpu.SemaphoreType.DMA,
        ],
    )
    def gather_k(idx_hbm, table_hbm, out_hbm, idx_v, rows_v, sem):
        wid = lax.axis_index("s") * NC + lax.axis_index("c")
        base = wid * rows_per_w
        pltpu.sync_copy(idx_hbm.at[pl.ds(base, rows_per_w)], idx_v)
        pltpu.async_copy(table_hbm.at[idx_v], rows_v, sem).wait()
        pltpu.sync_copy(rows_v, out_hbm.at[pl.ds(base, rows_per_w)])

    return gather_k(idx_pad, emb_table)


def _tc_body(C, L, D, TPAD,
             x_ref, rows_ref, mf_ref, gamma_ref, beta_ref, bias_ref, w_ref,
             out_ref, m_ref, vt_ref):
    @pl.when(pl.program_id(0) == 0)
    def _init():
        rows = rows_ref[...]
        mu = jnp.mean(rows, axis=1, keepdims=True)
        xc = rows - mu
        var = jnp.mean(xc * xc, axis=1, keepdims=True)
        ln = xc * lax.rsqrt(var + EPS) * gamma_ref[...] + beta_ref[...]
        mf = mf_ref[...]
        lnm = ln * mf
        t_col = lax.broadcasted_iota(jnp.int32, (C, TPAD), 1) // L
        c_row = lax.broadcasted_iota(jnp.int32, (C, TPAD), 0)
        sel = jnp.where(t_col == c_row, 1.0, 0.0)
        pool = lax.dot(sel, lnm, preferred_element_type=jnp.float32)
        den = lax.dot(sel, mf, preferred_element_type=jnp.float32)
        col = pool / den
        a_mat = lax.dot_general(col, w_ref[...], (((1,), (1,)), ((), ())),
                                preferred_element_type=jnp.float32)
        v = lax.dot_general(bias_ref[...], w_ref[...], (((1,), (1,)), ((), ())),
                            preferred_element_type=jnp.float32)
        a_tiled = pltpu.repeat(a_mat, C, axis=1)
        j_blk = lax.broadcasted_iota(jnp.int32, (C, C * D), 1) // D
        c_blk = lax.broadcasted_iota(jnp.int32, (C, C * D), 0)
        m_ref[...] = jnp.where(j_blk == c_blk, a_tiled, 0.0)
        vt_ref[...] = pltpu.repeat(v, C, axis=1)

    out_ref[...] = (lax.dot(x_ref[...], m_ref[...],
                            preferred_element_type=jnp.float32)
                    + vt_ref[...])


def kernel(x_num, num_col_input_ids, num_att_mask, emb_table, ln_gamma,
           ln_beta, num_bias, W_align):
    B, C = x_num.shape
    _, L = num_col_input_ids.shape
    D = emb_table.shape[1]
    T = C * L
    TPAD = ((T + 8 * NW - 1) // (8 * NW)) * (8 * NW)

    idx_pad = jnp.zeros((TPAD,), jnp.int32).at[:T].set(
        num_col_input_ids.reshape(-1))
    rows = _sc_gather(idx_pad, emb_table, D)

    mf_pad = jnp.zeros((TPAD, 1), jnp.float32).at[:T, :].set(
        num_att_mask.astype(jnp.float32).reshape(T, 1))

    BB = 256
    NBLK = B // BB
    out_flat = pl.pallas_call(
        functools.partial(_tc_body, C, L, D, TPAD),
        grid=(NBLK,),
        in_specs=[
            pl.BlockSpec((BB, C), lambda i: (i, 0)),
            pl.BlockSpec((TPAD, D), lambda i: (0, 0)),
            pl.BlockSpec((TPAD, 1), lambda i: (0, 0)),
            pl.BlockSpec((1, D), lambda i: (0, 0)),
            pl.BlockSpec((1, D), lambda i: (0, 0)),
            pl.BlockSpec((1, D), lambda i: (0, 0)),
            pl.BlockSpec((D, D), lambda i: (0, 0)),
        ],
        out_specs=pl.BlockSpec((BB, C * D), lambda i: (i, 0)),
        out_shape=jax.ShapeDtypeStruct((B, C * D), jnp.float32),
        scratch_shapes=[
            pltpu.VMEM((C, C * D), jnp.float32),
            pltpu.VMEM((1, C * D), jnp.float32),
        ],
    )(x_num, rows, mf_pad, ln_gamma.reshape(1, D), ln_beta.reshape(1, D),
      num_bias.reshape(1, D), W_align)

    out = out_flat.reshape(B, C, D)
    attention_mask = jnp.ones((B, C), dtype=jnp.float32)
    return out, attention_mask
